# baseline (speedup 1.0000x reference)
import functools

import jax
import jax.numpy as jnp
from jax import lax
from jax.experimental import pallas as pl
from jax.experimental.pallas import tpu as pltpu
from jax.experimental.pallas import tpu_sc as plsc

_BATCH = 16384
_NCHUNK = 2
_CB = _BATCH // _NCHUNK
_IN = 256
_HID = 256
_NSLOT = 24
_K = 6
_TILE = 4096

_SEL = tuple(j for j in range(24) if j != 0 and j != 12)
_SENTINEL = -2147483648
_NCORE = 2
_NSUB = 16
_NW = _NCORE * _NSUB
_CHUNK = _CB // _NW
_LANES = 16
_GROUPS = _CHUNK // _LANES


def _mlp_body(x_ref, w1_ref, b1_ref, w2_ref, b2_ref, w3_ref, b3_ref, o_ref):
    x = x_ref[...]
    h = jnp.dot(x, w1_ref[...], preferred_element_type=jnp.float32) + b1_ref[...]
    h = jnp.maximum(h, 0.0)
    h = jnp.dot(h, w2_ref[...], preferred_element_type=jnp.float32) + b2_ref[...]
    h = jnp.maximum(h, 0.0)
    logits = jnp.dot(h, w3_ref[...], preferred_element_type=jnp.float32) + b3_ref[...]
    o_ref[...] = jnp.transpose(logits)


def _tc_logits_t(x, W1, b1, W2, b2, W3, b3):
    return pl.pallas_call(
        _mlp_body,
        grid=(_CB // _TILE,),
        in_specs=[
            pl.BlockSpec((_TILE, _IN), lambda i: (i, 0)),
            pl.BlockSpec((_IN, _HID), lambda i: (0, 0)),
            pl.BlockSpec((1, _HID), lambda i: (0, 0)),
            pl.BlockSpec((_HID, _HID), lambda i: (0, 0)),
            pl.BlockSpec((1, _HID), lambda i: (0, 0)),
            pl.BlockSpec((_HID, _NSLOT), lambda i: (0, 0)),
            pl.BlockSpec((1, _NSLOT), lambda i: (0, 0)),
        ],
        out_specs=pl.BlockSpec((_NSLOT, _TILE), lambda i: (0, i)),
        out_shape=jax.ShapeDtypeStruct((_NSLOT, _CB), jnp.float32),
    )(x, W1, b1, W2, b2, W3, b3)


def _tree_max(vals):
    while len(vals) > 1:
        nxt = [jnp.maximum(vals[i], vals[i + 1]) for i in range(0, len(vals) - 1, 2)]
        if len(vals) % 2:
            nxt.append(vals[-1])
        vals = nxt
    return vals[0]


def _sc_mask_body(logits_hbm, out_hbm, in_v, out_v):
    wid = lax.axis_index("s") * _NCORE + lax.axis_index("c")
    base = wid * _CHUNK
    pltpu.sync_copy(logits_hbm.at[:, pl.ds(base, _CHUNK)], in_v)

    def group(g, carry):
        r0 = g * _LANES
        ones = jnp.ones((_LANES,), jnp.float32)
        keys = []
        for j in _SEL:
            v = in_v[j, pl.ds(r0, _LANES)]
            u = plsc.bitcast(v, jnp.int32)
            k = u ^ ((u >> 31) & jnp.int32(0x7FFFFFFF))
            k = (k & jnp.int32(-32)) | jnp.int32(31 - j)
            keys.append(k)
        accs = [None] * len(_SEL)
        sentinel = jnp.int32(_SENTINEL)
        for _ in range(_K):
            m = _tree_max(keys)
            for t in range(len(_SEL)):
                pick = keys[t] == m
                sel_f = jnp.where(pick, 1.0, 0.0)
                accs[t] = sel_f if accs[t] is None else jnp.maximum(accs[t], sel_f)
                keys[t] = jnp.where(pick, sentinel, keys[t])
        out_v[0, pl.ds(r0, _LANES)] = ones
        out_v[12, pl.ds(r0, _LANES)] = ones
        for t, j in enumerate(_SEL):
            out_v[j, pl.ds(r0, _LANES)] = accs[t]
        return carry

    lax.fori_loop(0, _GROUPS, group, 0)
    pltpu.sync_copy(out_v, out_hbm.at[:, pl.ds(base, _CHUNK)])


_sc_mask = functools.partial(
    pl.kernel,
    out_type=jax.ShapeDtypeStruct((_NSLOT, _CB), jnp.float32),
    mesh=plsc.VectorSubcoreMesh(
        core_axis_name="c", subcore_axis_name="s",
        num_cores=_NCORE, num_subcores=_NSUB),
    scratch_types=[
        pltpu.VMEM((_NSLOT, _CHUNK), jnp.float32),
        pltpu.VMEM((_NSLOT, _CHUNK), jnp.float32),
    ],
    compiler_params=pltpu.CompilerParams(needs_layout_passes=False),
)(_sc_mask_body)


@jax.jit
def kernel(qoi_features, W1, b1, W2, b2, W3, b3):
    b1r = b1.reshape(1, _HID)
    b2r = b2.reshape(1, _HID)
    b3r = b3.reshape(1, _NSLOT)
    masks = []
    logits = [
        _tc_logits_t(qoi_features[c * _CB:(c + 1) * _CB], W1, b1r, W2, b2r, W3, b3r)
        for c in range(_NCHUNK)
    ]
    masks = [_sc_mask(lt) for lt in logits]
    mask_t = jnp.concatenate(masks, axis=1)
    return mask_t.T.reshape(_BATCH, 2, 12)


# speedup vs baseline: 2.7906x; 2.7906x over previous
import jax
import jax.numpy as jnp
from jax.experimental import pallas as pl

_BATCH = 16384
_IN = 256
_HID = 256
_NSLOT = 24
_K = 6
_TILE = 2048

_SENTINEL = -2147483648


def _body(x_ref, w1_ref, b1_ref, w2_ref, b2_ref, w3_ref, b3_ref, o_ref):
    x = x_ref[...]
    h = jnp.dot(x, w1_ref[...], preferred_element_type=jnp.float32) + b1_ref[...]
    h = jnp.maximum(h, 0.0)
    h = jnp.dot(h, w2_ref[...], preferred_element_type=jnp.float32) + b2_ref[...]
    h = jnp.maximum(h, 0.0)
    logits = jnp.dot(h, w3_ref[...], preferred_element_type=jnp.float32) + b3_ref[...]

    lt = jnp.transpose(logits)
    row = jax.lax.broadcasted_iota(jnp.int32, lt.shape, 0)
    selectable = (row != 0) & (row != 12)
    u = lt.view(jnp.int32)
    k = u ^ ((u >> 31) & jnp.int32(0x7FFFFFFF))
    k = (k & jnp.int32(-32)) | (jnp.int32(31) - row)
    work = jnp.where(selectable, k, jnp.int32(_SENTINEL))
    acc = jnp.where(selectable, 0.0, 1.0)
    for _ in range(_K):
        m = jnp.max(work, axis=0, keepdims=True)
        pick = work == m
        acc = jnp.where(pick, 1.0, acc)
        work = jnp.where(pick, jnp.int32(_SENTINEL), work)
    o_ref[...] = acc


@jax.jit
def kernel(qoi_features, W1, b1, W2, b2, W3, b3):
    out = pl.pallas_call(
        _body,
        grid=(_BATCH // _TILE,),
        in_specs=[
            pl.BlockSpec((_TILE, _IN), lambda i: (i, 0)),
            pl.BlockSpec((_IN, _HID), lambda i: (0, 0)),
            pl.BlockSpec((1, _HID), lambda i: (0, 0)),
            pl.BlockSpec((_HID, _HID), lambda i: (0, 0)),
            pl.BlockSpec((1, _HID), lambda i: (0, 0)),
            pl.BlockSpec((_HID, _NSLOT), lambda i: (0, 0)),
            pl.BlockSpec((1, _NSLOT), lambda i: (0, 0)),
        ],
        out_specs=pl.BlockSpec((_NSLOT, _TILE), lambda i: (0, i)),
        out_shape=jax.ShapeDtypeStruct((_NSLOT, _BATCH), jnp.float32),
    )(qoi_features, W1, b1.reshape(1, _HID), W2, b2.reshape(1, _HID),
      W3, b3.reshape(1, _NSLOT))
    return out.T.reshape(_BATCH, 2, 12)


# speedup vs baseline: 2.9671x; 1.0633x over previous
import jax
import jax.numpy as jnp
from jax.experimental import pallas as pl

_BATCH = 16384
_IN = 256
_HID = 256
_NSLOT = 24
_K = 6
_TILE = 4096

_SENTINEL = -2147483648


def _body(x_ref, w1_ref, b1_ref, w2_ref, b2_ref, w3_ref, b3_ref, o_ref):
    x = x_ref[...]
    h = jnp.dot(x, w1_ref[...], preferred_element_type=jnp.float32) + b1_ref[...]
    h = jnp.maximum(h, 0.0)
    h = jnp.dot(h, w2_ref[...], preferred_element_type=jnp.float32) + b2_ref[...]
    h = jnp.maximum(h, 0.0)
    logits = jnp.dot(h, w3_ref[...], preferred_element_type=jnp.float32) + b3_ref[...]

    lt = jnp.transpose(logits)
    row = jax.lax.broadcasted_iota(jnp.int32, lt.shape, 0)
    selectable = (row != 0) & (row != 12)
    u = lt.view(jnp.int32)
    k = u ^ ((u >> 31) & jnp.int32(0x7FFFFFFF))
    k = (k & jnp.int32(-32)) | (jnp.int32(31) - row)
    work = jnp.where(selectable, k, jnp.int32(_SENTINEL))
    acc = jnp.where(selectable, 0.0, 1.0)
    for _ in range(_K):
        m = jnp.max(work, axis=0, keepdims=True)
        pick = work == m
        acc = jnp.where(pick, 1.0, acc)
        work = jnp.where(pick, jnp.int32(_SENTINEL), work)
    o_ref[...] = acc


@jax.jit
def kernel(qoi_features, W1, b1, W2, b2, W3, b3):
    out = pl.pallas_call(
        _body,
        grid=(_BATCH // _TILE,),
        in_specs=[
            pl.BlockSpec((_TILE, _IN), lambda i: (i, 0)),
            pl.BlockSpec((_IN, _HID), lambda i: (0, 0)),
            pl.BlockSpec((1, _HID), lambda i: (0, 0)),
            pl.BlockSpec((_HID, _HID), lambda i: (0, 0)),
            pl.BlockSpec((1, _HID), lambda i: (0, 0)),
            pl.BlockSpec((_HID, _NSLOT), lambda i: (0, 0)),
            pl.BlockSpec((1, _NSLOT), lambda i: (0, 0)),
        ],
        out_specs=pl.BlockSpec((_NSLOT, _TILE), lambda i: (0, i)),
        out_shape=jax.ShapeDtypeStruct((_NSLOT, _BATCH), jnp.float32),
    )(qoi_features, W1, b1.reshape(1, _HID), W2, b2.reshape(1, _HID),
      W3, b3.reshape(1, _NSLOT))
    return out.T.reshape(_BATCH, 2, 12)


# speedup vs baseline: 3.2859x; 1.1074x over previous
import jax
import jax.numpy as jnp
from jax.experimental import pallas as pl

_BATCH = 16384
_IN = 256
_HID = 256
_NSLOT = 24
_K = 6
_TILE = 4096

_SENTINEL = -2147483648


def _body(x_ref, w1_ref, b1_ref, w2_ref, b2_ref, w3_ref, b3_ref, o_ref):
    x = x_ref[...]
    h = jnp.dot(x, w1_ref[...], preferred_element_type=jnp.float32) + b1_ref[...]
    h = jnp.maximum(h, 0.0)
    h = jnp.dot(h, w2_ref[...], preferred_element_type=jnp.float32) + b2_ref[...]
    h = jnp.maximum(h, 0.0)
    lt = jax.lax.dot_general(
        w3_ref[...], h, (((1,), (1,)), ((), ())),
        preferred_element_type=jnp.float32) + b3_ref[...]
    row = jax.lax.broadcasted_iota(jnp.int32, lt.shape, 0)
    selectable = (row != 0) & (row != 12)
    u = lt.view(jnp.int32)
    k = u ^ ((u >> 31) & jnp.int32(0x7FFFFFFF))
    k = (k & jnp.int32(-32)) | (jnp.int32(31) - row)
    work = jnp.where(selectable, k, jnp.int32(_SENTINEL))
    acc = jnp.where(selectable, 0.0, 1.0)
    for _ in range(_K):
        m = jnp.max(work, axis=0, keepdims=True)
        pick = work == m
        acc = jnp.where(pick, 1.0, acc)
        work = jnp.where(pick, jnp.int32(_SENTINEL), work)
    o_ref[...] = acc


@jax.jit
def kernel(qoi_features, W1, b1, W2, b2, W3, b3):
    out = pl.pallas_call(
        _body,
        grid=(_BATCH // _TILE,),
        in_specs=[
            pl.BlockSpec((_TILE, _IN), lambda i: (i, 0)),
            pl.BlockSpec((_IN, _HID), lambda i: (0, 0)),
            pl.BlockSpec((1, _HID), lambda i: (0, 0)),
            pl.BlockSpec((_HID, _HID), lambda i: (0, 0)),
            pl.BlockSpec((1, _HID), lambda i: (0, 0)),
            pl.BlockSpec((_NSLOT, _HID), lambda i: (0, 0)),
            pl.BlockSpec((_NSLOT, 1), lambda i: (0, 0)),
        ],
        out_specs=pl.BlockSpec((_NSLOT, _TILE), lambda i: (0, i)),
        out_shape=jax.ShapeDtypeStruct((_NSLOT, _BATCH), jnp.float32),
    )(qoi_features, W1, b1.reshape(1, _HID), W2, b2.reshape(1, _HID),
      W3.T, b3.reshape(_NSLOT, 1))
    return out.T.reshape(_BATCH, 2, 12)


# speedup vs baseline: 3.2894x; 1.0010x over previous
import jax
import jax.numpy as jnp
from jax.experimental import pallas as pl
from jax.experimental.pallas import tpu as pltpu

_BATCH = 16384
_IN = 256
_HID = 256
_NSLOT = 24
_K = 6
_TILE = 4096

_SENTINEL = -2147483648


def _body(x_ref, w1_ref, b1_ref, w2_ref, b2_ref, w3_ref, b3_ref, o_ref):
    x = x_ref[...]
    h = jnp.dot(x, w1_ref[...], preferred_element_type=jnp.float32) + b1_ref[...]
    h = jnp.maximum(h, 0.0)
    h = jnp.dot(h, w2_ref[...], preferred_element_type=jnp.float32) + b2_ref[...]
    h = jnp.maximum(h, 0.0)
    lt = jax.lax.dot_general(
        w3_ref[...], h, (((1,), (1,)), ((), ())),
        preferred_element_type=jnp.float32) + b3_ref[...]
    row = jax.lax.broadcasted_iota(jnp.int32, lt.shape, 0)
    selectable = (row != 0) & (row != 12)
    u = lt.view(jnp.int32)
    k = u ^ ((u >> 31) & jnp.int32(0x7FFFFFFF))
    k = (k & jnp.int32(-32)) | (jnp.int32(31) - row)
    work = jnp.where(selectable, k, jnp.int32(_SENTINEL))
    acc = jnp.where(selectable, 0.0, 1.0)
    for _ in range(_K):
        m = jnp.max(work, axis=0, keepdims=True)
        pick = work == m
        acc = jnp.where(pick, 1.0, acc)
        work = jnp.where(pick, jnp.int32(_SENTINEL), work)
    o_ref[...] = acc


@jax.jit
def kernel(qoi_features, W1, b1, W2, b2, W3, b3):
    out = pl.pallas_call(
        _body,
        grid=(_BATCH // _TILE,),
        in_specs=[
            pl.BlockSpec((_TILE, _IN), lambda i: (i, 0)),
            pl.BlockSpec((_IN, _HID), lambda i: (0, 0)),
            pl.BlockSpec((1, _HID), lambda i: (0, 0)),
            pl.BlockSpec((_HID, _HID), lambda i: (0, 0)),
            pl.BlockSpec((1, _HID), lambda i: (0, 0)),
            pl.BlockSpec((_NSLOT, _HID), lambda i: (0, 0)),
            pl.BlockSpec((_NSLOT, 1), lambda i: (0, 0)),
        ],
        out_specs=pl.BlockSpec((_NSLOT, _TILE), lambda i: (0, i)),
        out_shape=jax.ShapeDtypeStruct((_NSLOT, _BATCH), jnp.float32),
        compiler_params=pltpu.CompilerParams(
            dimension_semantics=("arbitrary",),
            vmem_limit_bytes=100 * 1024 * 1024,
        ),
    )(qoi_features, W1, b1.reshape(1, _HID), W2, b2.reshape(1, _HID),
      W3.T, b3.reshape(_NSLOT, 1))
    return out.T.reshape(_BATCH, 2, 12)
